# baseline (speedup 1.0000x reference)
import jax
import jax.numpy as jnp
from jax.experimental import pallas as pl
from jax.experimental.pallas import tpu as pltpu

NUM_GATES = 8
CAPACITY_FACTOR = 1.25
MIN_EXPERT_CAPACITY = 4
TBLK = 256


def _route_kernel(eb_ref, out_d_ref, out_c_ref, *, cap):
    b = pl.program_id(0)
    tb = pl.program_id(1)
    e = eb_ref[b]
    t0 = tb * TBLK
    shape = (TBLK, NUM_GATES, cap)
    t = jax.lax.broadcasted_iota(jnp.int32, shape, 0) + t0
    g = jax.lax.broadcasted_iota(jnp.int32, shape, 1)
    c = jax.lax.broadcasted_iota(jnp.int32, shape, 2)
    val = jnp.where((t < cap) & (g == e) & (c == t), 1.0, 0.0).astype(jnp.float32)
    out_d_ref[0] = val
    out_c_ref[0] = val


def kernel(x, current_y):
    b, gs, _ = x.shape
    cap = int(gs * CAPACITY_FACTOR / NUM_GATES)
    cap = max(min(gs, cap), MIN_EXPERT_CAPACITY)

    eb = jnp.remainder(current_y.astype(jnp.int32), NUM_GATES)

    import functools
    kern = functools.partial(_route_kernel, cap=cap)
    grid_spec = pltpu.PrefetchScalarGridSpec(
        num_scalar_prefetch=1,
        grid=(b, gs // TBLK),
        in_specs=[],
        out_specs=[
            pl.BlockSpec((1, TBLK, NUM_GATES, cap), lambda i, j, eb_ref: (i, j, 0, 0)),
            pl.BlockSpec((1, TBLK, NUM_GATES, cap), lambda i, j, eb_ref: (i, j, 0, 0)),
        ],
    )
    out_shape = [
        jax.ShapeDtypeStruct((b, gs, NUM_GATES, cap), jnp.float32),
        jax.ShapeDtypeStruct((b, gs, NUM_GATES, cap), jnp.float32),
    ]
    dispatch, combine = pl.pallas_call(
        kern, grid_spec=grid_spec, out_shape=out_shape
    )(eb)
    return dispatch, combine


# speedup vs baseline: 1.4180x; 1.4180x over previous
import jax
import jax.numpy as jnp
from jax.experimental import pallas as pl
from jax.experimental.pallas import tpu as pltpu

NUM_GATES = 8
CAPACITY_FACTOR = 1.25
MIN_EXPERT_CAPACITY = 4
TBLK = 256


def _route_kernel(eb_ref, out_ref, *, cap, k_total):
    b = pl.program_id(0)
    tb = pl.program_id(1)
    e = eb_ref[b]
    t0 = tb * TBLK
    t = jax.lax.broadcasted_iota(jnp.int32, (TBLK, k_total), 0) + t0
    k = jax.lax.broadcasted_iota(jnp.int32, (TBLK, k_total), 1)
    val = jnp.where((t < cap) & (k == e * cap + t), 1.0, 0.0).astype(jnp.float32)
    out_ref[0] = val


def kernel(x, current_y):
    b, gs, _ = x.shape
    cap = int(gs * CAPACITY_FACTOR / NUM_GATES)
    cap = max(min(gs, cap), MIN_EXPERT_CAPACITY)
    k_total = NUM_GATES * cap

    eb = jnp.remainder(current_y.astype(jnp.int32), NUM_GATES)

    import functools
    kern = functools.partial(_route_kernel, cap=cap, k_total=k_total)
    grid_spec = pltpu.PrefetchScalarGridSpec(
        num_scalar_prefetch=1,
        grid=(b, gs // TBLK),
        in_specs=[],
        out_specs=[
            pl.BlockSpec((1, TBLK, k_total), lambda i, j, eb_ref: (i, j, 0)),
        ],
    )
    out_shape = [
        jax.ShapeDtypeStruct((b, gs, k_total), jnp.float32),
    ]
    (out,) = pl.pallas_call(
        kern, grid_spec=grid_spec, out_shape=out_shape
    )(eb)
    out = out.reshape(b, gs, NUM_GATES, cap)
    return out, out


# speedup vs baseline: 1.4273x; 1.0066x over previous
import jax
import jax.numpy as jnp
from jax.experimental import pallas as pl
from jax.experimental.pallas import tpu as pltpu

NUM_GATES = 8
CAPACITY_FACTOR = 1.25
MIN_EXPERT_CAPACITY = 4
TBLK = 1024


def _route_kernel(eb_ref, out_ref, *, cap, k_total):
    b = pl.program_id(0)
    tb = pl.program_id(1)
    e = eb_ref[b]
    t0 = tb * TBLK
    t = jax.lax.broadcasted_iota(jnp.int32, (TBLK, k_total), 0) + t0
    k = jax.lax.broadcasted_iota(jnp.int32, (TBLK, k_total), 1)
    val = jnp.where((t < cap) & (k == e * cap + t), 1.0, 0.0).astype(jnp.float32)
    out_ref[0] = val


def kernel(x, current_y):
    b, gs, _ = x.shape
    cap = int(gs * CAPACITY_FACTOR / NUM_GATES)
    cap = max(min(gs, cap), MIN_EXPERT_CAPACITY)
    k_total = NUM_GATES * cap

    eb = jnp.remainder(current_y.astype(jnp.int32), NUM_GATES)

    import functools
    kern = functools.partial(_route_kernel, cap=cap, k_total=k_total)
    grid_spec = pltpu.PrefetchScalarGridSpec(
        num_scalar_prefetch=1,
        grid=(b, gs // TBLK),
        in_specs=[],
        out_specs=[
            pl.BlockSpec((1, TBLK, k_total), lambda i, j, eb_ref: (i, j, 0)),
        ],
    )
    out_shape = [
        jax.ShapeDtypeStruct((b, gs, k_total), jnp.float32),
    ]
    (out,) = pl.pallas_call(
        kern, grid_spec=grid_spec, out_shape=out_shape
    )(eb)
    out = out.reshape(b, gs, NUM_GATES, cap)
    return out, out


# speedup vs baseline: 1.4316x; 1.0031x over previous
import jax
import jax.numpy as jnp
from jax.experimental import pallas as pl
from jax.experimental.pallas import tpu as pltpu

NUM_GATES = 8
CAPACITY_FACTOR = 1.25
MIN_EXPERT_CAPACITY = 4
TBLK = 1024


def _route_kernel(eb_ref, out_ref, *, cap, k_total):
    b = pl.program_id(0)
    tb = pl.program_id(1)
    e = eb_ref[b]
    t0 = tb * TBLK
    t = jax.lax.broadcasted_iota(jnp.int32, (TBLK, k_total), 0) + t0
    k = jax.lax.broadcasted_iota(jnp.int32, (TBLK, k_total), 1)
    val = jnp.where((t < cap) & (k == e * cap + t), 1.0, 0.0).astype(jnp.float32)
    out_ref[0] = val


def kernel(x, current_y):
    b, gs, _ = x.shape
    cap = int(gs * CAPACITY_FACTOR / NUM_GATES)
    cap = max(min(gs, cap), MIN_EXPERT_CAPACITY)
    k_total = NUM_GATES * cap

    eb = jnp.remainder(current_y.astype(jnp.int32), NUM_GATES)

    import functools
    kern = functools.partial(_route_kernel, cap=cap, k_total=k_total)
    grid_spec = pltpu.PrefetchScalarGridSpec(
        num_scalar_prefetch=1,
        grid=(b, gs // TBLK),
        in_specs=[],
        out_specs=[
            pl.BlockSpec((1, TBLK, k_total), lambda i, j, eb_ref: (i, j, 0)),
        ],
    )
    out_shape = [
        jax.ShapeDtypeStruct((b, gs, k_total), jnp.float32),
    ]
    (out,) = pl.pallas_call(
        kern, grid_spec=grid_spec, out_shape=out_shape,
        compiler_params=pltpu.CompilerParams(
            dimension_semantics=("parallel", "parallel")),
    )(eb)
    out = out.reshape(b, gs, NUM_GATES, cap)
    return out, out
